# Initial kernel scaffold; baseline (speedup 1.0000x reference)
import jax
import jax.numpy as jnp
from jax.experimental import pallas as pl


def kernel(x0, output_coords, coords, W, sigma_d, kappa_vm, indices0, neighbor_idx):
    raise NotImplementedError("write your pallas kernel here")


# speedup vs baseline: 24.2521x; 24.2521x over previous
import functools

import jax
import jax.numpy as jnp
from jax import lax
from jax.experimental import pallas as pl
from jax.experimental.pallas import tpu as pltpu
from jax.experimental.pallas import tpu_sc as plsc

MODEL_DIM = 128
OUT_DIM = 128
KN = 4
NC = 2
NS = 16
NW = NC * NS
L = 16
C = 128
CD = 16

_MESH = plsc.VectorSubcoreMesh(core_axis_name="c", subcore_axis_name="s",
                               num_cores=NC, num_subcores=NS)
_SC_PARAMS = pltpu.CompilerParams(use_tc_tiling_on_sc=False,
                                  needs_layout_passes=False)


def _proj_body(x_ref, w_ref, o_ref):
    o_ref[...] = lax.dot_general(
        x_ref[...], w_ref[...], (((1,), (1,)), ((), ())),
        preferred_element_type=jnp.float32)


def _project(x2d, W, block_rows):
    n = x2d.shape[0]
    return pl.pallas_call(
        _proj_body,
        grid=(n // block_rows,),
        in_specs=[
            pl.BlockSpec((block_rows, MODEL_DIM), lambda i: (i, 0)),
            pl.BlockSpec((OUT_DIM, MODEL_DIM), lambda i: (0, 0)),
        ],
        out_specs=pl.BlockSpec((block_rows, OUT_DIM), lambda i: (i, 0)),
        out_shape=jax.ShapeDtypeStruct((n, OUT_DIM), jnp.float32),
    )(x2d, W)


def _worker_id():
    return lax.axis_index("s") * NC + lax.axis_index("c")


def _rsqrt(x):
    i = plsc.bitcast(x, jnp.int32)
    i = 0x5F3759DF - lax.shift_right_arithmetic(i, 1)
    y = plsc.bitcast(i, jnp.float32)
    for _ in range(3):
        y = y * (1.5 - 0.5 * x * y * y)
    return y


def _nodexy_body(ind0_hbm, cpad_hbm, nxy_hbm, idx_v, pr_v, sem):
    wid = _worker_id()
    nchunks = nxy_hbm.shape[0] // C

    def body(t, carry):
        i = wid + t * NW

        @pl.when(i < nchunks)
        def _():
            base = i * C
            pltpu.sync_copy(ind0_hbm.at[pl.ds(base, C)], idx_v)
            pltpu.async_copy(cpad_hbm.at[idx_v], pr_v, sem).wait()
            pltpu.sync_copy(pr_v, nxy_hbm.at[pl.ds(base, C)])

        return carry

    lax.fori_loop(0, pl.cdiv(nchunks, NW), body, 0)


def _build_nodexy(ind0_pad, cpad):
    npad = ind0_pad.shape[0]
    f = pl.kernel(
        _nodexy_body,
        out_type=jax.ShapeDtypeStruct((npad, CD), jnp.float32),
        mesh=_MESH,
        scratch_types=[
            pltpu.VMEM((C,), jnp.int32),
            pltpu.VMEM((C, CD), jnp.float32),
            pltpu.SemaphoreType.DMA,
        ],
        compiler_params=_SC_PARAMS,
    )
    return f(ind0_pad, cpad)


def _interp_body(x_hbm, nbr_hbm, nxy_hbm, ocx_hbm, ocy_hbm, nis_hbm, kap_hbm,
                 out_hbm,
                 nbr_v, nxy_v, ocx_v, ocy_v, w_v, rows_v, out_v, cst_v,
                 sem_c, sem_f):
    wid = _worker_id()
    nchunks = out_hbm.shape[0] // C

    pltpu.sync_copy(nis_hbm, cst_v.at[0])
    pltpu.sync_copy(kap_hbm, cst_v.at[1])
    nis = cst_v[0, :]
    kap = cst_v[1, :]
    iot = lax.iota(jnp.int32, L)
    col0 = jnp.full((L,), 0, jnp.int32)
    col1 = jnp.full((L,), 1, jnp.int32)

    def chunk(t, carry):
        i = wid + t * NW

        @pl.when(i < nchunks)
        def _():
            base = i * C
            pltpu.sync_copy(nbr_hbm.at[pl.ds(KN * i, KN)], nbr_v)
            hf = [pltpu.async_copy(x_hbm.at[nbr_v.at[j]],
                                   rows_v.at[pl.ds(j * C, C)], sem_f)
                  for j in range(KN)]
            hc = [pltpu.async_copy(nxy_hbm.at[nbr_v.at[j]], nxy_v.at[j], sem_c)
                  for j in range(KN)]
            pltpu.sync_copy(ocx_hbm.at[pl.ds(base, C)], ocx_v)
            pltpu.sync_copy(ocy_hbm.at[pl.ds(base, C)], ocy_v)
            for h in hc:
                h.wait()

            for g in range(C // L):
                p0 = g * L
                ocx = ocx_v[pl.ds(p0, L)]
                ocy = ocy_v[pl.ds(p0, L)]
                logits = []
                for k in range(KN):
                    r = (KN * p0 + k) + KN * iot
                    rj = lax.shift_right_arithmetic(r, 7)
                    rm = r & 127
                    ncx = plsc.load_gather(nxy_v, [rj, rm, col0])
                    ncy = plsc.load_gather(nxy_v, [rj, rm, col1])
                    dx = ncx - ocx
                    dy = ncy - ocy
                    d2 = dx * dx + dy * dy
                    rs = _rsqrt(jnp.maximum(d2, 1e-20))
                    logits.append(nis * d2 + kap * (dx * rs))
                m = jnp.maximum(jnp.maximum(logits[0], logits[1]),
                                jnp.maximum(logits[2], logits[3]))
                es = [jnp.exp(lg - m) for lg in logits]
                inv = 1.0 / (es[0] + es[1] + es[2] + es[3])
                for k in range(KN):
                    r = (KN * p0 + k) + KN * iot
                    plsc.store_scatter(w_v, [r], es[k] * inv)

            for h in hf:
                h.wait()

            def pt(p, cc):
                r0 = KN * p
                acc = [None] * (OUT_DIM // L)
                for k in range(KN):
                    wv = plsc.load_gather(
                        w_v, [jnp.full((L,), r0 + k, jnp.int32)])
                    for j in range(OUT_DIM // L):
                        v = rows_v[r0 + k, pl.ds(j * L, L)]
                        acc[j] = wv * v if k == 0 else acc[j] + wv * v
                for j in range(OUT_DIM // L):
                    out_v[p, pl.ds(j * L, L)] = acc[j]
                return cc

            lax.fori_loop(0, C, pt, 0)
            pltpu.sync_copy(out_v, out_hbm.at[pl.ds(base, C)])

        return carry

    lax.fori_loop(0, pl.cdiv(nchunks, NW), chunk, 0)


def _interpolate(x, nbr2, nxy, ocx, ocy, nis, kap, npad_out):
    f = pl.kernel(
        _interp_body,
        out_type=jax.ShapeDtypeStruct((npad_out, OUT_DIM), jnp.float32),
        mesh=_MESH,
        scratch_types=[
            pltpu.VMEM((KN, C), jnp.int32),
            pltpu.VMEM((KN, C, CD), jnp.float32),
            pltpu.VMEM((C,), jnp.float32),
            pltpu.VMEM((C,), jnp.float32),
            pltpu.VMEM((KN * C,), jnp.float32),
            pltpu.VMEM((KN * C, OUT_DIM), jnp.float32),
            pltpu.VMEM((C, OUT_DIM), jnp.float32),
            pltpu.VMEM((2, L), jnp.float32),
            pltpu.SemaphoreType.DMA,
            pltpu.SemaphoreType.DMA,
        ],
        compiler_params=_SC_PARAMS,
    )
    return f(x, nbr2, nxy, ocx, ocy, nis, kap)


def kernel(x0, output_coords, coords, W, sigma_d, kappa_vm, indices0,
           neighbor_idx):
    b = x0.shape[0]
    n_nodes = x0.shape[1]
    n_out = neighbor_idx.shape[1]
    npad_out = pl.cdiv(n_out, C) * C
    npad_nodes = pl.cdiv(n_nodes, C) * C

    x = _project(x0.reshape(n_nodes, MODEL_DIM), W, 1000)

    ind0 = indices0.reshape(-1).astype(jnp.int32)
    ind0 = jnp.pad(ind0, (0, npad_nodes - n_nodes))
    cpad = jnp.concatenate(
        [coords.T, jnp.zeros((coords.shape[1], CD - 2), jnp.float32)], axis=1)
    nbr = neighbor_idx.reshape(-1).astype(jnp.int32)
    nbr = jnp.pad(nbr, (0, (npad_out - n_out) * KN))
    nbr2 = nbr.reshape(npad_out * KN // C, C)
    ocx = jnp.pad(output_coords[0].reshape(-1), (0, npad_out - n_out))
    ocy = jnp.pad(output_coords[1].reshape(-1), (0, npad_out - n_out))
    nis = jnp.broadcast_to(
        (-1.0 / (2.0 * sigma_d * sigma_d)).astype(jnp.float32), (L,))
    kap = jnp.broadcast_to(kappa_vm.astype(jnp.float32), (L,))

    nxy = _build_nodexy(ind0, cpad)
    out2d = _interpolate(x, nbr2, nxy, ocx, ocy, nis, kap, npad_out)
    return out2d[:n_out].reshape(b, n_out, OUT_DIM)


# speedup vs baseline: 30.3970x; 1.2534x over previous
import functools

import jax
import jax.numpy as jnp
from jax import lax
from jax.experimental import pallas as pl
from jax.experimental.pallas import tpu as pltpu
from jax.experimental.pallas import tpu_sc as plsc

MODEL_DIM = 128
OUT_DIM = 128
KN = 4
NC = 2
NS = 16
NW = NC * NS
L = 16
C = 128
CA = 1024
CS = 256
CB = 64
CD = 16

_MESH = plsc.VectorSubcoreMesh(core_axis_name="c", subcore_axis_name="s",
                               num_cores=NC, num_subcores=NS)
_SC_PARAMS = pltpu.CompilerParams(use_tc_tiling_on_sc=False,
                                  needs_layout_passes=False)
_SC_PARAMS_TILED = pltpu.CompilerParams(use_tc_tiling_on_sc=True,
                                        needs_layout_passes=False)


def _proj_body(x_ref, w_ref, o_ref):
    o_ref[...] = lax.dot_general(
        x_ref[0], w_ref[...], (((1,), (1,)), ((), ())),
        preferred_element_type=jnp.float32)


def _project(x0, W, block_rows):
    n = x0.shape[1]
    return pl.pallas_call(
        _proj_body,
        grid=(n // block_rows,),
        in_specs=[
            pl.BlockSpec((1, block_rows, MODEL_DIM), lambda i: (0, i, 0)),
            pl.BlockSpec((OUT_DIM, MODEL_DIM), lambda i: (0, 0)),
        ],
        out_specs=pl.BlockSpec((block_rows, OUT_DIM), lambda i: (i, 0)),
        out_shape=jax.ShapeDtypeStruct((n, OUT_DIM), jnp.float32),
    )(x0, W)


def _worker_id():
    return lax.axis_index("s") * NC + lax.axis_index("c")


def _rsqrt(x):
    i = plsc.bitcast(x, jnp.int32)
    i = 0x5F3759DF - lax.shift_right_arithmetic(i, 1)
    y = plsc.bitcast(i, jnp.float32)
    for _ in range(3):
        y = y * (1.5 - 0.5 * x * y * y)
    return y


def _nodexy_body(ind0_hbm, cpad_hbm, nxy_hbm, idx_v, pr_v, sem):
    wid = _worker_id()
    nchunks = nxy_hbm.shape[0] // CA
    nrow = CA // 128

    def body(t, carry):
        i = wid + t * NW

        @pl.when(i < nchunks)
        def _():
            pltpu.sync_copy(ind0_hbm.at[pl.ds(nrow * i, nrow)], idx_v)
            hs = [pltpu.async_copy(cpad_hbm.at[idx_v.at[j]],
                                   pr_v.at[pl.ds(j * 128, 128)], sem)
                  for j in range(nrow)]
            for h in hs:
                h.wait()
            pltpu.sync_copy(pr_v, nxy_hbm.at[pl.ds(CA * i, CA)])

        return carry

    lax.fori_loop(0, pl.cdiv(nchunks, NW), body, 0)


def _build_nodexy(ind0_2d, cpad, npad_nodes):
    f = pl.kernel(
        _nodexy_body,
        out_type=jax.ShapeDtypeStruct((npad_nodes, CD), jnp.float32),
        mesh=_MESH,
        scratch_types=[
            pltpu.VMEM((CA // 128, 128), jnp.int32),
            pltpu.VMEM((CA, CD), jnp.float32),
            pltpu.SemaphoreType.DMA,
        ],
        compiler_params=_SC_PARAMS,
    )
    return f(ind0_2d, cpad)


def _make_weights_body(n_out):
    nmain = n_out // C
    tail = n_out - nmain * C

    def body(nbr_hbm, nxy_hbm, ocx_hbm, ocy_hbm, nis_hbm, kap_hbm, w_hbm,
             nbr_v, nxy_v, ocx_v, ocy_v, w_v, cst_v, sem):
        wid = _worker_id()
        pltpu.sync_copy(nis_hbm, cst_v.at[0])
        pltpu.sync_copy(kap_hbm, cst_v.at[1])
        nis = cst_v[0, :]
        kap = cst_v[1, :]
        iot = lax.iota(jnp.int32, L)
        col0 = jnp.full((L,), 0, jnp.int32)
        col1 = jnp.full((L,), 1, jnp.int32)

        def emit(base, row0, npts):
            nrows = KN * npts // 128
            pltpu.sync_copy(nbr_hbm.at[pl.ds(row0, nrows)],
                            nbr_v.at[pl.ds(0, nrows)])
            hc = [pltpu.async_copy(nxy_hbm.at[nbr_v.at[j]], nxy_v.at[j], sem)
                  for j in range(nrows)]
            pltpu.sync_copy(ocx_hbm.at[pl.ds(base, npts)],
                            ocx_v.at[pl.ds(0, npts)])
            pltpu.sync_copy(ocy_hbm.at[pl.ds(base, npts)],
                            ocy_v.at[pl.ds(0, npts)])
            for h in hc:
                h.wait()
            for g in range(npts // L):
                p0 = g * L
                ocx = ocx_v[pl.ds(p0, L)]
                ocy = ocy_v[pl.ds(p0, L)]
                logits = []
                for k in range(KN):
                    r = (KN * p0 + k) + KN * iot
                    rj = lax.shift_right_arithmetic(r, 7)
                    rm = r & 127
                    ncx = plsc.load_gather(nxy_v, [rj, rm, col0])
                    ncy = plsc.load_gather(nxy_v, [rj, rm, col1])
                    dx = ncx - ocx
                    dy = ncy - ocy
                    d2 = dx * dx + dy * dy
                    rs = _rsqrt(jnp.maximum(d2, 1e-20))
                    logits.append(nis * d2 + kap * (dx * rs))
                m = jnp.maximum(jnp.maximum(logits[0], logits[1]),
                                jnp.maximum(logits[2], logits[3]))
                es = [jnp.exp(lg - m) for lg in logits]
                inv = 1.0 / (es[0] + es[1] + es[2] + es[3])
                for k in range(KN):
                    r = (KN * p0 + k) + KN * iot
                    rj = lax.shift_right_arithmetic(r, 7)
                    rm = r & 127
                    plsc.store_scatter(w_v, [rj, rm], es[k] * inv)
            pltpu.sync_copy(w_v.at[pl.ds(0, nrows)],
                            w_hbm.at[pl.ds(row0, nrows)])

        def loop(t, carry):
            i = wid + t * NW

            @pl.when(i < nmain)
            def _():
                emit(C * i, KN * i, C)

            return carry

        lax.fori_loop(0, pl.cdiv(nmain, NW), loop, 0)
        if tail:
            @pl.when(wid == NW - 1)
            def _():
                emit(nmain * C, KN * nmain, tail)

    return body


def _weights(nbr2, nxy, ocx, ocy, nis, kap):
    n_out = ocx.shape[0]
    f = pl.kernel(
        _make_weights_body(n_out),
        out_type=jax.ShapeDtypeStruct(nbr2.shape, jnp.float32),
        mesh=_MESH,
        scratch_types=[
            pltpu.VMEM((KN, 128), jnp.int32),
            pltpu.VMEM((KN, 128, CD), jnp.float32),
            pltpu.VMEM((C,), jnp.float32),
            pltpu.VMEM((C,), jnp.float32),
            pltpu.VMEM((KN, 128), jnp.float32),
            pltpu.VMEM((2, L), jnp.float32),
            pltpu.SemaphoreType.DMA,
        ],
        compiler_params=_SC_PARAMS,
    )
    return f(nbr2, nxy, ocx, ocy, nis, kap)


def _make_combine_body(n_out):
    nsup = n_out // CS
    tail = n_out - nsup * CS
    nsub = CS // CB
    rsub = KN * CB // 128

    def body(x_hbm, nbr_hbm, w_hbm, out_hbm, nbr_v, w_v, rows_v, out_v, sem):
        wid = _worker_id()
        iot = lax.iota(jnp.int32, L)

        def fire(slot, s):
            for j in range(rsub):
                pltpu.async_copy(x_hbm.at[nbr_v.at[rsub * s + j]],
                                 rows_v.at[slot, pl.ds(128 * j, 128)], sem)

        def drain(slot, s):
            for j in range(rsub):
                pltpu.make_async_copy(x_hbm.at[nbr_v.at[rsub * s + j]],
                                      rows_v.at[slot, pl.ds(128 * j, 128)],
                                      sem).wait()

        def combine(slot, base, npts, woff):
            def pt(p, cc):
                r0 = KN * p
                acc = [None] * (OUT_DIM // L)
                for k in range(KN):
                    rf = jnp.full((L,), woff + r0 + k, jnp.int32)
                    wv = plsc.load_gather(
                        w_v, [lax.shift_right_arithmetic(rf, 7), rf & 127])
                    for j in range(OUT_DIM // L):
                        v = rows_v[slot, r0 + k, pl.ds(j * L, L)]
                        acc[j] = wv * v if k == 0 else acc[j] + wv * v
                for j in range(OUT_DIM // L):
                    out_v[p, pl.ds(j * L, L)] = acc[j]
                return cc

            lax.fori_loop(0, npts, pt, 0)
            pltpu.sync_copy(out_v.at[pl.ds(0, npts)],
                            out_hbm.at[0, pl.ds(base, npts)])

        def super_chunk(i):
            row0 = pl.multiple_of((KN * CS // 128) * i, 8)
            pltpu.sync_copy(nbr_hbm.at[pl.ds(row0, KN * CS // 128)], nbr_v)
            pltpu.sync_copy(w_hbm.at[pl.ds(row0, KN * CS // 128)], w_v)
            fire(0, 0)
            for s in range(nsub):
                if s + 1 < nsub:
                    fire((s + 1) % 2, s + 1)
                drain(s % 2, s)
                combine(s % 2, pl.multiple_of(CS * i + CB * s, 8), CB,
                        KN * CB * s)

        def loop(t, carry):
            i = wid + t * NW

            @pl.when(i < nsup)
            def _():
                super_chunk(i)

            return carry

        lax.fori_loop(0, pl.cdiv(nsup, NW), loop, 0)

        if tail:
            @pl.when(wid == NW - 1)
            def _():
                row0 = KN * nsup * CS // 128
                pltpu.sync_copy(nbr_hbm.at[pl.ds(row0, KN * CS // 128)],
                                nbr_v)
                pltpu.sync_copy(w_hbm.at[pl.ds(row0, KN * CS // 128)], w_v)
                off = 0
                while off < tail:
                    npts = min(CB, tail - off)
                    local0 = KN * off // 128
                    trows = KN * npts // 128
                    hs = [pltpu.async_copy(x_hbm.at[nbr_v.at[local0 + j]],
                                           rows_v.at[0, pl.ds(128 * j, 128)],
                                           sem)
                          for j in range(trows)]
                    for h in hs:
                        h.wait()
                    combine(0, nsup * CS + off, npts, KN * off)
                    off += npts

    return body


def _combine(x, nbr2, w2, n_out):
    f = pl.kernel(
        _make_combine_body(n_out),
        out_type=jax.ShapeDtypeStruct((1, n_out, OUT_DIM), jnp.float32),
        mesh=_MESH,
        scratch_types=[
            pltpu.VMEM((KN * CS // 128, 128), jnp.int32),
            pltpu.VMEM((KN * CS // 128, 128), jnp.float32),
            pltpu.VMEM((2, KN * CB, OUT_DIM), jnp.float32),
            pltpu.VMEM((CB, OUT_DIM), jnp.float32),
            pltpu.SemaphoreType.DMA,
        ],
        compiler_params=_SC_PARAMS_TILED,
    )
    return f(x, nbr2, w2)


def kernel(x0, output_coords, coords, W, sigma_d, kappa_vm, indices0,
           neighbor_idx):
    n_nodes = x0.shape[1]
    n_out = neighbor_idx.shape[1]
    npad_nodes = pl.cdiv(n_nodes, CA) * CA

    x = _project(x0, W, 2000)

    ind0 = indices0.reshape(-1).astype(jnp.int32)
    ind0_2d = jnp.pad(ind0, (0, npad_nodes - n_nodes)).reshape(-1, 128)
    cpad = jnp.concatenate(
        [coords.T, jnp.zeros((coords.shape[1], CD - 2), jnp.float32)], axis=1)
    nbr_rows = pl.cdiv(n_out * KN // 128, 8) * 8
    nbr2 = jnp.pad(neighbor_idx.reshape(-1).astype(jnp.int32),
                   (0, nbr_rows * 128 - n_out * KN)).reshape(-1, 128)
    ocx = output_coords[0].reshape(-1)
    ocy = output_coords[1].reshape(-1)
    nis = jnp.broadcast_to(
        (-1.0 / (2.0 * sigma_d * sigma_d)).astype(jnp.float32), (L,))
    kap = jnp.broadcast_to(kappa_vm.astype(jnp.float32), (L,))

    nxy = _build_nodexy(ind0_2d, cpad, npad_nodes)
    w2 = _weights(nbr2, nxy, ocx, ocy, nis, kap)
    return _combine(x, nbr2, w2, n_out)


# speedup vs baseline: 49.1385x; 1.6166x over previous
import functools

import jax
import jax.numpy as jnp
from jax import lax
from jax.experimental import pallas as pl
from jax.experimental.pallas import tpu as pltpu
from jax.experimental.pallas import tpu_sc as plsc

MODEL_DIM = 128
OUT_DIM = 128
KN = 4
NC = 2
NS = 16
NW = NC * NS
L = 16
C = 128
C0 = 800
CA = 1024
CS = 256
CB = 64
CD = 16

_MESH = plsc.VectorSubcoreMesh(core_axis_name="c", subcore_axis_name="s",
                               num_cores=NC, num_subcores=NS)
_SC_PARAMS = pltpu.CompilerParams(use_tc_tiling_on_sc=False,
                                  needs_layout_passes=False)
_SC_PARAMS_TILED = pltpu.CompilerParams(use_tc_tiling_on_sc=True,
                                        needs_layout_passes=False)


def _proj_body(x_ref, w_ref, o_ref):
    o_ref[...] = lax.dot_general(
        x_ref[0], w_ref[...], (((1,), (1,)), ((), ())),
        preferred_element_type=jnp.float32)


def _project(x0, W, block_rows):
    n = x0.shape[1]
    return pl.pallas_call(
        _proj_body,
        grid=(n // block_rows,),
        in_specs=[
            pl.BlockSpec((1, block_rows, MODEL_DIM), lambda i: (0, i, 0)),
            pl.BlockSpec((OUT_DIM, MODEL_DIM), lambda i: (0, 0)),
        ],
        out_specs=pl.BlockSpec((block_rows, OUT_DIM), lambda i: (i, 0)),
        out_shape=jax.ShapeDtypeStruct((n, OUT_DIM), jnp.float32),
    )(x0, W)


def _worker_id():
    return lax.axis_index("s") * NC + lax.axis_index("c")


def _rsqrt(x):
    i = plsc.bitcast(x, jnp.int32)
    i = 0x5F3759DF - lax.shift_right_arithmetic(i, 1)
    y = plsc.bitcast(i, jnp.float32)
    for _ in range(3):
        y = y * (1.5 - 0.5 * x * y * y)
    return y


def _coords16_body(coords_hbm, c16_hbm, cx_v, cy_v, pr_v, sem):
    wid = _worker_id()
    n = c16_hbm.shape[0]
    nchunks = n // C0
    iot = lax.iota(jnp.int32, L)
    col0 = jnp.full((L,), 0, jnp.int32)
    col1 = jnp.full((L,), 1, jnp.int32)

    def body(t, carry):
        i = wid + t * NW

        @pl.when(i < nchunks)
        def _():
            base = i * C0
            h0 = pltpu.async_copy(coords_hbm.at[0, pl.ds(base, C0)], cx_v, sem)
            h1 = pltpu.async_copy(coords_hbm.at[1, pl.ds(base, C0)], cy_v, sem)
            h0.wait()
            h1.wait()
            for g in range(C0 // L):
                p = g * L + iot
                plsc.store_scatter(pr_v, [p, col0], cx_v[pl.ds(g * L, L)])
                plsc.store_scatter(pr_v, [p, col1], cy_v[pl.ds(g * L, L)])
            pltpu.sync_copy(pr_v, c16_hbm.at[pl.ds(base, C0)])

        return carry

    lax.fori_loop(0, pl.cdiv(nchunks, NW), body, 0)


def _coords16(coords):
    n = coords.shape[1]
    f = pl.kernel(
        _coords16_body,
        out_type=jax.ShapeDtypeStruct((n, CD), jnp.float32),
        mesh=_MESH,
        scratch_types=[
            pltpu.VMEM((C0,), jnp.float32),
            pltpu.VMEM((C0,), jnp.float32),
            pltpu.VMEM((C0, CD), jnp.float32),
            pltpu.SemaphoreType.DMA,
        ],
        compiler_params=_SC_PARAMS,
    )
    return f(coords)


def _nodexy_body(ind0_hbm, c16_hbm, nxy_hbm, idx_v, pr_v, sem):
    wid = _worker_id()
    nchunks = nxy_hbm.shape[0] // CA
    nrow = CA // 128

    def body(t, carry):
        i = wid + t * NW

        @pl.when(i < nchunks)
        def _():
            pltpu.sync_copy(ind0_hbm.at[pl.ds(nrow * i, nrow)], idx_v)
            hs = [pltpu.async_copy(c16_hbm.at[idx_v.at[j]],
                                   pr_v.at[pl.ds(j * 128, 128)], sem)
                  for j in range(nrow)]
            for h in hs:
                h.wait()
            pltpu.sync_copy(pr_v, nxy_hbm.at[pl.ds(CA * i, CA)])

        return carry

    lax.fori_loop(0, pl.cdiv(nchunks, NW), body, 0)


def _build_nodexy(ind0_2d, c16, npad_nodes):
    f = pl.kernel(
        _nodexy_body,
        out_type=jax.ShapeDtypeStruct((npad_nodes, CD), jnp.float32),
        mesh=_MESH,
        scratch_types=[
            pltpu.VMEM((CA // 128, 128), jnp.int32),
            pltpu.VMEM((CA, CD), jnp.float32),
            pltpu.SemaphoreType.DMA,
        ],
        compiler_params=_SC_PARAMS,
    )
    return f(ind0_2d, c16)


def _make_weights_body(n_out):
    nmain = n_out // C
    tail = n_out - nmain * C

    def body(nbrt_hbm, nxy_hbm, ocx_hbm, ocy_hbm, nis_hbm, kap_hbm,
             w_hbm, nbrg_hbm,
             nbr_v, nxy_v, ocx_v, ocy_v, w_v, cst_v, sem_n, sem_g):
        wid = _worker_id()
        pltpu.sync_copy(nis_hbm, cst_v.at[0])
        pltpu.sync_copy(kap_hbm, cst_v.at[1])
        nis = cst_v[0, :]
        kap = cst_v[1, :]
        iot = lax.iota(jnp.int32, L)
        col0 = jnp.full((L,), 0, jnp.int32)
        col1 = jnp.full((L,), 1, jnp.int32)

        def emit(base, row0, npts):
            hn = [pltpu.async_copy(nbrt_hbm.at[k, pl.ds(base, npts)],
                                   nbr_v.at[k, pl.ds(0, npts)], sem_n)
                  for k in range(KN)]
            ho = [pltpu.async_copy(ocx_hbm.at[pl.ds(base, npts)],
                                   ocx_v.at[pl.ds(0, npts)], sem_g),
                  pltpu.async_copy(ocy_hbm.at[pl.ds(base, npts)],
                                   ocy_v.at[pl.ds(0, npts)], sem_g)]
            for h in hn:
                h.wait()
            hg = [pltpu.async_copy(nxy_hbm.at[nbr_v.at[k, pl.ds(0, npts)]],
                                   nxy_v.at[k, pl.ds(0, npts)], sem_g)
                  for k in range(KN)]
            for h in ho + hg:
                h.wait()
            for g in range(npts // L):
                p = g * L + iot
                ocx = ocx_v[pl.ds(g * L, L)]
                ocy = ocy_v[pl.ds(g * L, L)]
                logits = []
                for k in range(KN):
                    kf = jnp.full((L,), k, jnp.int32)
                    ncx = plsc.load_gather(nxy_v, [kf, p, col0])
                    ncy = plsc.load_gather(nxy_v, [kf, p, col1])
                    dx = ncx - ocx
                    dy = ncy - ocy
                    d2 = dx * dx + dy * dy
                    rs = _rsqrt(jnp.maximum(d2, 1e-20))
                    logits.append(nis * d2 + kap * (dx * rs))
                m = jnp.maximum(jnp.maximum(logits[0], logits[1]),
                                jnp.maximum(logits[2], logits[3]))
                es = [jnp.exp(lg - m) for lg in logits]
                inv = 1.0 / (es[0] + es[1] + es[2] + es[3])
                for k in range(KN):
                    kf = jnp.full((L,), k, jnp.int32)
                    plsc.store_scatter(w_v, [kf, p], es[k] * inv)
            pltpu.sync_copy(w_v, w_hbm.at[pl.ds(row0, KN)])
            pltpu.sync_copy(nbr_v, nbrg_hbm.at[pl.ds(row0, KN)])

        def loop(t, carry):
            i = wid + t * NW

            @pl.when(i < nmain)
            def _():
                emit(C * i, KN * i, C)

            return carry

        lax.fori_loop(0, pl.cdiv(nmain, NW), loop, 0)
        if tail:
            @pl.when(wid == NW - 1)
            def _():
                emit(nmain * C, KN * nmain, tail)

    return body


def _weights(nbrt, nxy, ocx, ocy, nis, kap, nbr_rows):
    n_out = ocx.shape[0]
    f = pl.kernel(
        _make_weights_body(n_out),
        out_type=(jax.ShapeDtypeStruct((nbr_rows, 128), jnp.float32),
                  jax.ShapeDtypeStruct((nbr_rows, 128), jnp.int32)),
        mesh=_MESH,
        scratch_types=[
            pltpu.VMEM((KN, 128), jnp.int32),
            pltpu.VMEM((KN, 128, CD), jnp.float32),
            pltpu.VMEM((C,), jnp.float32),
            pltpu.VMEM((C,), jnp.float32),
            pltpu.VMEM((KN, 128), jnp.float32),
            pltpu.VMEM((2, L), jnp.float32),
            pltpu.SemaphoreType.DMA,
            pltpu.SemaphoreType.DMA,
        ],
        compiler_params=_SC_PARAMS,
    )
    return f(nbrt, nxy, ocx, ocy, nis, kap)


def _make_combine_body(n_out):
    nsup = n_out // CS
    tail = n_out - nsup * CS
    nsub = CS // CB
    nrow_sup = KN * CS // 128

    def body(x_hbm, nbrg_hbm, w_hbm, out_hbm,
             nbr_v, w_v, rows_v, out_v, sem_f, sem_io):
        wid = _worker_id()

        def fire(slot, s, npts=CB):
            lc, h = s // 2, s % 2
            for k in range(KN):
                pltpu.async_copy(
                    x_hbm.at[nbr_v.at[KN * lc + k, pl.ds(CB * h, npts)]],
                    rows_v.at[slot, pl.ds(CB * k, npts)], sem_f)

        def drain(slot, s, npts=CB):
            lc, h = s // 2, s % 2
            for k in range(KN):
                pltpu.make_async_copy(
                    x_hbm.at[nbr_v.at[KN * lc + k, pl.ds(CB * h, npts)]],
                    rows_v.at[slot, pl.ds(CB * k, npts)], sem_f).wait()

        def combine(slot, s, base, npts):
            lc, h = s // 2, s % 2

            def pt(p, cc):
                acc = [None] * (OUT_DIM // L)
                for k in range(KN):
                    wv = plsc.load_gather(
                        w_v, [jnp.full((L,), KN * lc + k, jnp.int32),
                              jnp.full((L,), CB * h, jnp.int32) + p])
                    for j in range(OUT_DIM // L):
                        v = rows_v[slot, CB * k + p, pl.ds(j * L, L)]
                        acc[j] = wv * v if k == 0 else acc[j] + wv * v
                for j in range(OUT_DIM // L):
                    out_v[slot, p, pl.ds(j * L, L)] = acc[j]
                return cc

            lax.fori_loop(0, npts, pt, 0, unroll=2)
            return pltpu.async_copy(
                out_v.at[slot, pl.ds(0, npts)],
                out_hbm.at[0, pl.ds(base, npts)], sem_io)

        def super_chunk(i):
            row0 = pl.multiple_of(nrow_sup * i, 8)
            hn = pltpu.async_copy(nbrg_hbm.at[pl.ds(row0, nrow_sup)], nbr_v,
                                  sem_io)
            hw = pltpu.async_copy(w_hbm.at[pl.ds(row0, nrow_sup)], w_v,
                                  sem_io)
            hn.wait()
            hw.wait()
            fire(0, 0)
            houts = []
            for s in range(nsub):
                if s + 1 < nsub:
                    fire((s + 1) % 2, s + 1)
                drain(s % 2, s)
                if s >= 2:
                    houts[s - 2].wait()
                houts.append(
                    combine(s % 2, s, pl.multiple_of(CS * i + CB * s, 8), CB))
            houts[-2].wait()
            houts[-1].wait()

        def loop(t, carry):
            i = wid + t * NW

            @pl.when(i < nsup)
            def _():
                super_chunk(i)

            return carry

        lax.fori_loop(0, pl.cdiv(nsup, NW), loop, 0)

        if tail:
            @pl.when(wid == NW - 1)
            def _():
                row0 = KN * nsup * CS // 128
                pltpu.sync_copy(nbrg_hbm.at[pl.ds(row0, nrow_sup)], nbr_v)
                pltpu.sync_copy(w_hbm.at[pl.ds(row0, nrow_sup)], w_v)
                off = 0
                while off < tail:
                    npts = min(CB, tail - off)
                    s = off // CB
                    fire(0, s, npts)
                    drain(0, s, npts)
                    combine(0, s, nsup * CS + off, npts).wait()
                    off += npts

    return body


def _combine(x, nbrg, w2, n_out):
    f = pl.kernel(
        _make_combine_body(n_out),
        out_type=jax.ShapeDtypeStruct((1, n_out, OUT_DIM), jnp.float32),
        mesh=_MESH,
        scratch_types=[
            pltpu.VMEM((KN * CS // 128, 128), jnp.int32),
            pltpu.VMEM((KN * CS // 128, 128), jnp.float32),
            pltpu.VMEM((2, KN * CB, OUT_DIM), jnp.float32),
            pltpu.VMEM((2, CB, OUT_DIM), jnp.float32),
            pltpu.SemaphoreType.DMA,
            pltpu.SemaphoreType.DMA,
        ],
        compiler_params=_SC_PARAMS_TILED,
    )
    return f(x, nbrg, w2)


def kernel(x0, output_coords, coords, W, sigma_d, kappa_vm, indices0,
           neighbor_idx):
    n_nodes = x0.shape[1]
    n_out = neighbor_idx.shape[1]
    npad_nodes = pl.cdiv(n_nodes, CA) * CA
    nbr_rows = pl.cdiv(n_out * KN // 128, 8) * 8

    x = _project(x0, W, 2000)

    ind0 = indices0.reshape(-1).astype(jnp.int32)
    ind0_2d = jnp.pad(ind0, (0, npad_nodes - n_nodes)).reshape(-1, 128)
    nbrt = jnp.transpose(neighbor_idx.astype(jnp.int32).reshape(n_out, KN))
    ocx = output_coords[0].reshape(-1)
    ocy = output_coords[1].reshape(-1)
    nis = jnp.broadcast_to(
        (-1.0 / (2.0 * sigma_d * sigma_d)).astype(jnp.float32), (L,))
    kap = jnp.broadcast_to(kappa_vm.astype(jnp.float32), (L,))

    c16 = _coords16(coords)
    nxy = _build_nodexy(ind0_2d, c16, npad_nodes)
    w2, nbrg = _weights(nbrt, nxy, ocx, ocy, nis, kap, nbr_rows)
    return _combine(x, nbrg, w2, n_out)
